# baseline (speedup 1.0000x reference)
import functools
import jax
import jax.numpy as jnp
from jax import lax
from jax.experimental import pallas as pl
from jax.experimental.pallas import tpu as pltpu
from jax.experimental.pallas import tpu_sc as plsc

_S = 8448
_C = 128
_NBUF = 4
_CB = 4
_SNB = 2


def _tc_kernel(x_ref, pe_ref, o_ref, ibuf, obuf, sem_in, sem_out):
    L, D = x_ref.shape[1], x_ref.shape[2]
    nchunk = _S // _C
    pos = pe_ref[:L, :][None, :, :]

    def in_cp(i):
        s = i % _NBUF
        return pltpu.make_async_copy(
            x_ref.at[pl.ds(i * _C, _C)], ibuf.at[s], sem_in.at[s])

    def out_cp(i):
        s = i % _NBUF
        return pltpu.make_async_copy(
            obuf.at[s], o_ref.at[pl.ds(i * _C, _C)], sem_out.at[s])

    for i in range(min(_NBUF, nchunk)):
        in_cp(i).start()
    for i in range(nchunk):
        s = i % _NBUF
        in_cp(i).wait()
        if i >= _NBUF:
            out_cp(i - _NBUF).wait()
        obuf[s] = ibuf[s] + pos
        out_cp(i).start()
        if i + _NBUF < nchunk:
            in_cp(i + _NBUF).start()
    for i in range(max(0, nchunk - _NBUF), nchunk):
        out_cp(i).wait()


def _tc_call(x, pe):
    B, L, D = x.shape
    return pl.pallas_call(
        _tc_kernel,
        in_specs=[
            pl.BlockSpec(memory_space=pltpu.MemorySpace.HBM),
            pl.BlockSpec(memory_space=pltpu.MemorySpace.VMEM),
        ],
        out_specs=pl.BlockSpec(memory_space=pltpu.MemorySpace.HBM),
        out_shape=jax.ShapeDtypeStruct((_S, L, D), x.dtype),
        scratch_shapes=[
            pltpu.VMEM((_NBUF, _C, L, D), x.dtype),
            pltpu.VMEM((_NBUF, _C, L, D), x.dtype),
            pltpu.SemaphoreType.DMA((_NBUF,)),
            pltpu.SemaphoreType.DMA((_NBUF,)),
        ],
        compiler_params=pltpu.CompilerParams(
            vmem_limit_bytes=100 * 1024 * 1024,
        ),
    )(x, pe)


def _sc_call(x, pe):
    B, L, D = x.shape
    info = plsc.get_sparse_core_info()
    NC, NS, LN = info.num_cores, info.num_subcores, info.num_lanes
    NW = NC * NS
    rows = B - _S
    rows_w = rows // NW
    nchunk = rows_w // _CB

    @functools.partial(
        pl.kernel,
        mesh=plsc.VectorSubcoreMesh(core_axis_name="c", subcore_axis_name="s"),
        out_type=jax.ShapeDtypeStruct((rows, L, D), jnp.float32),
        scratch_types=[
            pltpu.VMEM((pe.shape[0], D), jnp.float32),
            pltpu.VMEM((_CB, L, D), jnp.float32),
            pltpu.VMEM((_CB, L, D), jnp.float32),
            pltpu.VMEM((_CB, L, D), jnp.float32),
            pltpu.VMEM((_CB, L, D), jnp.float32),
            pltpu.SemaphoreType.DMA,
            pltpu.SemaphoreType.DMA,
            pltpu.SemaphoreType.DMA,
            pltpu.SemaphoreType.DMA,
        ],
    )
    def _k(x_hbm, pe_hbm, out_hbm, pe_v, i0, i1, o0, o1, si0, si1, so0, so1):
        wid = lax.axis_index("s") * NC + lax.axis_index("c")
        base = wid * rows_w
        pltpu.sync_copy(pe_hbm, pe_v)
        ibufs, obufs = (i0, i1), (o0, o1)
        sin, sout = (si0, si1), (so0, so1)

        def in_cp(i, b):
            return pltpu.make_async_copy(
                x_hbm.at[pl.ds(_S + base + i * _CB, _CB)], ibufs[b], sin[b])

        def out_cp(i, b):
            return pltpu.make_async_copy(
                obufs[b], out_hbm.at[pl.ds(base + i * _CB, _CB)], sout[b])

        in_cp(0, 0).start()
        in_cp(1, 1).start()

        def outer(g, _):
            for b in range(_SNB):
                i = g * _SNB + b
                in_cp(i, b).wait()

                @pl.when(i >= _SNB)
                def _():
                    out_cp(i - _SNB, b).wait()

                ib, ob = ibufs[b], obufs[b]

                def row_body(r, _):
                    for bb in range(_CB):
                        for l in range(D // LN):
                            sl = pl.ds(l * LN, LN)
                            ob[bb, r, sl] = ib[bb, r, sl] + pe_v[r, sl]
                    return 0

                lax.fori_loop(0, L, row_body, 0)
                out_cp(i, b).start()

                @pl.when(i + _SNB < nchunk)
                def _():
                    in_cp(i + _SNB, b).start()
            return 0

        lax.fori_loop(0, nchunk // _SNB, outer, 0)
        out_cp(nchunk - 2, 0).wait()
        out_cp(nchunk - 1, 1).wait()

    return _k(x, pe)


def kernel(x, pe):
    sc_out = _sc_call(x, pe)
    tc_out = _tc_call(x, pe)
    return jnp.concatenate([tc_out, sc_out], axis=0)


# speedup vs baseline: 1.3774x; 1.3774x over previous
import functools
import jax
import jax.numpy as jnp
from jax import lax
from jax.experimental import pallas as pl
from jax.experimental.pallas import tpu as pltpu
from jax.experimental.pallas import tpu_sc as plsc

_CB = 4
_NBUF = 2


def kernel(x, pe):
    B, L, D = x.shape
    info = plsc.get_sparse_core_info()
    NC, NS, LN = info.num_cores, info.num_subcores, info.num_lanes
    NW = NC * NS
    rows_w = B // NW
    nchunk = rows_w // _CB

    @functools.partial(
        pl.kernel,
        mesh=plsc.VectorSubcoreMesh(core_axis_name="c", subcore_axis_name="s"),
        out_type=jax.ShapeDtypeStruct((B, L, D), jnp.float32),
        scratch_types=[
            pltpu.VMEM((pe.shape[0], D), jnp.float32),
            pltpu.VMEM((_CB, L, D), jnp.float32),
            pltpu.VMEM((_CB, L, D), jnp.float32),
            pltpu.VMEM((_CB, L, D), jnp.float32),
            pltpu.VMEM((_CB, L, D), jnp.float32),
            pltpu.SemaphoreType.DMA,
            pltpu.SemaphoreType.DMA,
            pltpu.SemaphoreType.DMA,
            pltpu.SemaphoreType.DMA,
        ],
    )
    def _k(x_hbm, pe_hbm, out_hbm, pe_v, i0, i1, o0, o1, si0, si1, so0, so1):
        wid = lax.axis_index("s") * NC + lax.axis_index("c")
        base = wid * rows_w
        pltpu.sync_copy(pe_hbm, pe_v)
        ibufs, obufs = (i0, i1), (o0, o1)
        sin, sout = (si0, si1), (so0, so1)

        def in_cp(i, b):
            return pltpu.make_async_copy(
                x_hbm.at[pl.ds(base + i * _CB, _CB)], ibufs[b], sin[b])

        def out_cp(i, b):
            return pltpu.make_async_copy(
                obufs[b], out_hbm.at[pl.ds(base + i * _CB, _CB)], sout[b])

        in_cp(0, 0).start()
        in_cp(1, 1).start()

        def outer(g, _):
            for b in range(_NBUF):
                i = g * _NBUF + b
                in_cp(i, b).wait()

                @pl.when(i >= _NBUF)
                def _():
                    out_cp(i - _NBUF, b).wait()

                ib, ob = ibufs[b], obufs[b]

                def row_body(r, _):
                    for bb in range(_CB):
                        for l in range(D // LN):
                            sl = pl.ds(l * LN, LN)
                            ob[bb, r, sl] = ib[bb, r, sl] + pe_v[r, sl]
                    return 0

                lax.fori_loop(0, L, row_body, 0)
                out_cp(i, b).start()

                @pl.when(i + _NBUF < nchunk)
                def _():
                    in_cp(i + _NBUF, b).start()
            return 0

        lax.fori_loop(0, nchunk // _NBUF, outer, 0)
        out_cp(nchunk - 2, 0).wait()
        out_cp(nchunk - 1, 1).wait()

    return _k(x, pe)
